# Initial kernel scaffold; baseline (speedup 1.0000x reference)
import jax
import jax.numpy as jnp
from jax.experimental import pallas as pl


def kernel(x, table0, table1, gamma, beta, W1, b1, W2, b2, W3, b3):
    raise NotImplementedError("write your pallas kernel here")


# speedup vs baseline: 54.3699x; 54.3699x over previous
import functools

import jax
import jax.numpy as jnp
from jax import lax
from jax.experimental import pallas as pl
from jax.experimental.pallas import tpu as pltpu
from jax.experimental.pallas import tpu_sc as plsc

B = 4096
F = 26
H = 20
VOCAB = 100000
EMB = 16
FEAT = F * EMB
NW = 32
ROWS = B * F
RPW = ROWS // NW
CHUNK = 128
NCHUNK = RPW // CHUNK
IPC = CHUNK * H


def _add_tables(t0, t1):
    g, r = 10, 1250

    def body(a_ref, b_ref, o_ref):
        o_ref[...] = a_ref[...] + b_ref[...]

    out = pl.pallas_call(
        body,
        grid=(g,),
        in_specs=[
            pl.BlockSpec((1, r, 128), lambda i: (i, 0, 0)),
            pl.BlockSpec((1, r, 128), lambda i: (i, 0, 0)),
        ],
        out_specs=pl.BlockSpec((1, r, 128), lambda i: (i, 0, 0)),
        out_shape=jax.ShapeDtypeStruct((g, r, 128), jnp.float32),
    )(t0.reshape(g, r, 128), t1.reshape(g, r, 128))
    return out.reshape(VOCAB, EMB)


def _make_emb_sum():
    mesh = plsc.VectorSubcoreMesh(core_axis_name="c", subcore_axis_name="s")

    @functools.partial(
        pl.kernel,
        mesh=mesh,
        compiler_params=pltpu.CompilerParams(use_tc_tiling_on_sc=False),
        out_type=jax.ShapeDtypeStruct((ROWS, EMB), jnp.float32),
        scratch_types=[
            pltpu.VMEM((H, CHUNK), jnp.int32),
            pltpu.VMEM((IPC, EMB), jnp.float32),
            pltpu.VMEM((CHUNK, EMB), jnp.float32),
            pltpu.SemaphoreType.DMA,
        ],
    )
    def emb_sum(tsum_hbm, idx_hbm, out_hbm, idx_v, rows_v, acc_v, sem):
        wid = lax.axis_index("s") * 2 + lax.axis_index("c")

        def chunk_body(c, carry):
            pltpu.sync_copy(idx_hbm.at[wid, c], idx_v)
            cps = [
                pltpu.async_copy(
                    tsum_hbm.at[idx_v.at[j]],
                    rows_v.at[pl.ds(j * CHUNK, CHUNK)],
                    sem,
                )
                for j in range(H)
            ]
            for cp in cps:
                cp.wait()

            def row_body(r, c2):
                base = r * H
                acc = rows_v[base, :]
                for h in range(1, H):
                    acc = acc + rows_v[base + h, :]
                acc_v[r, :] = acc
                return c2

            lax.fori_loop(0, CHUNK, row_body, 0)
            pltpu.sync_copy(
                acc_v, out_hbm.at[pl.ds(wid * RPW + c * CHUNK, CHUNK)]
            )
            return carry

        lax.fori_loop(0, NCHUNK, chunk_body, 0)

    return emb_sum


_emb_sum = _make_emb_sum()


def _mlp(s, gamma, beta, W1, b1, W2, b2, W3, b3):
    BB = 512

    def body(s_ref, g_ref, be_ref, w1_ref, b1_ref, w2_ref, b2_ref,
             w3_ref, b3_ref, o_ref):
        sb = s_ref[...]
        mean = jnp.mean(sb, axis=-1, keepdims=True)
        var = jnp.mean((sb - mean) ** 2, axis=-1, keepdims=True)
        hn = (sb - mean) * lax.rsqrt(var + 1e-5) * g_ref[...] + be_ref[...]
        h1 = jnp.maximum(jnp.dot(hn, w1_ref[...]) + b1_ref[...], 0.0)
        h2 = jnp.maximum(jnp.dot(h1, w2_ref[...]) + b2_ref[...], 0.0)
        o_ref[...] = jax.nn.sigmoid(jnp.dot(h2, w3_ref[...]) + b3_ref[...])

    full = lambda shape: pl.BlockSpec(shape, lambda i: tuple(0 for _ in shape))
    return pl.pallas_call(
        body,
        grid=(B // BB,),
        in_specs=[
            pl.BlockSpec((BB, FEAT), lambda i: (i, 0)),
            full((1, FEAT)),
            full((1, FEAT)),
            full((FEAT, 1024)),
            full((1, 1024)),
            full((1024, 512)),
            full((1, 512)),
            full((512, 1)),
            full((1, 1)),
        ],
        out_specs=pl.BlockSpec((BB, 1), lambda i: (i, 0)),
        out_shape=jax.ShapeDtypeStruct((B, 1), jnp.float32),
    )(s, gamma.reshape(1, FEAT), beta.reshape(1, FEAT), W1,
      b1.reshape(1, 1024), W2, b2.reshape(1, 512), W3, b3.reshape(1, 1))


def kernel(x, table0, table1, gamma, beta, W1, b1, W2, b2, W3, b3):
    idx = x.astype(jnp.int32).reshape(NW, NCHUNK, H, CHUNK)
    tsum = _add_tables(table0, table1)
    s = _emb_sum(tsum, idx).reshape(B, FEAT)
    return _mlp(s, gamma, beta, W1, b1, W2, b2, W3, b3)


# speedup vs baseline: 67.9138x; 1.2491x over previous
import functools

import jax
import jax.numpy as jnp
from jax import lax
from jax.experimental import pallas as pl
from jax.experimental.pallas import tpu as pltpu
from jax.experimental.pallas import tpu_sc as plsc

B = 4096
F = 26
H = 20
VOCAB = 100000
EMB = 16
FEAT = F * EMB
NW = 32
ROWS = B * F
RPW = ROWS // NW
CHUNK = 128
NCHUNK = RPW // CHUNK
IPC = CHUNK * H


def _add_tables(t0, t1):
    g, r = 10, 1250

    def body(a_ref, b_ref, o_ref):
        o_ref[...] = a_ref[...] + b_ref[...]

    out = pl.pallas_call(
        body,
        grid=(g,),
        in_specs=[
            pl.BlockSpec((1, r, 128), lambda i: (i, 0, 0)),
            pl.BlockSpec((1, r, 128), lambda i: (i, 0, 0)),
        ],
        out_specs=pl.BlockSpec((1, r, 128), lambda i: (i, 0, 0)),
        out_shape=jax.ShapeDtypeStruct((g, r, 128), jnp.float32),
    )(t0.reshape(g, r, 128), t1.reshape(g, r, 128))
    return out.reshape(VOCAB, EMB)


def _make_emb_sum():
    mesh = plsc.VectorSubcoreMesh(core_axis_name="c", subcore_axis_name="s")

    @functools.partial(
        pl.kernel,
        mesh=mesh,
        compiler_params=pltpu.CompilerParams(use_tc_tiling_on_sc=False),
        out_type=jax.ShapeDtypeStruct((ROWS, EMB), jnp.float32),
        scratch_types=[
            pltpu.VMEM((H, CHUNK), jnp.int32),
            pltpu.VMEM((H, CHUNK), jnp.int32),
            pltpu.VMEM((IPC, EMB), jnp.float32),
            pltpu.VMEM((IPC, EMB), jnp.float32),
            pltpu.VMEM((CHUNK, EMB), jnp.float32),
            pltpu.SemaphoreType.DMA,
            pltpu.SemaphoreType.DMA,
        ],
    )
    def emb_sum(tsum_hbm, idx_hbm, out_hbm, idx0, idx1, rows0, rows1,
                acc_v, sem0, sem1):
        wid = lax.axis_index("s") * 2 + lax.axis_index("c")
        idx_b = (idx0, idx1)
        rows_b = (rows0, rows1)
        sem_b = (sem0, sem1)

        def fire(c, slot):
            pltpu.sync_copy(idx_hbm.at[wid, c], idx_b[slot])
            for j in range(H):
                pltpu.async_copy(
                    tsum_hbm.at[idx_b[slot].at[j]],
                    rows_b[slot].at[pl.ds(j * CHUNK, CHUNK)],
                    sem_b[slot],
                )

        def drain_sum_write(c, slot):
            for j in range(H):
                pltpu.make_async_copy(
                    tsum_hbm.at[idx_b[slot].at[j]],
                    rows_b[slot].at[pl.ds(j * CHUNK, CHUNK)],
                    sem_b[slot],
                ).wait()
            rows_v = rows_b[slot]

            def row_body(r, c2):
                base = r * H
                acc = rows_v[base, :]
                for h in range(1, H):
                    acc = acc + rows_v[base + h, :]
                acc_v[r, :] = acc
                return c2

            lax.fori_loop(0, CHUNK, row_body, 0)
            pltpu.sync_copy(
                acc_v, out_hbm.at[pl.ds(wid * RPW + c * CHUNK, CHUNK)]
            )

        fire(0, 0)

        def pair_body(g, carry):
            c = 2 * g
            fire(c + 1, 1)
            drain_sum_write(c, 0)

            @pl.when(g + 1 < NCHUNK // 2)
            def _():
                fire(c + 2, 0)

            drain_sum_write(c + 1, 1)
            return carry

        lax.fori_loop(0, NCHUNK // 2, pair_body, 0)

    return emb_sum


_emb_sum = _make_emb_sum()


def _mlp(s, gamma, beta, W1, b1, W2, b2, W3, b3):
    BB = 512

    def body(s_ref, g_ref, be_ref, w1_ref, b1_ref, w2_ref, b2_ref,
             w3_ref, b3_ref, o_ref):
        sb = s_ref[...]
        mean = jnp.mean(sb, axis=-1, keepdims=True)
        var = jnp.mean((sb - mean) ** 2, axis=-1, keepdims=True)
        hn = (sb - mean) * lax.rsqrt(var + 1e-5) * g_ref[...] + be_ref[...]
        h1 = jnp.maximum(jnp.dot(hn, w1_ref[...]) + b1_ref[...], 0.0)
        h2 = jnp.maximum(jnp.dot(h1, w2_ref[...]) + b2_ref[...], 0.0)
        o_ref[...] = jax.nn.sigmoid(jnp.dot(h2, w3_ref[...]) + b3_ref[...])

    full = lambda shape: pl.BlockSpec(shape, lambda i: tuple(0 for _ in shape))
    return pl.pallas_call(
        body,
        grid=(B // BB,),
        in_specs=[
            pl.BlockSpec((BB, FEAT), lambda i: (i, 0)),
            full((1, FEAT)),
            full((1, FEAT)),
            full((FEAT, 1024)),
            full((1, 1024)),
            full((1024, 512)),
            full((1, 512)),
            full((512, 1)),
            full((1, 1)),
        ],
        out_specs=pl.BlockSpec((BB, 1), lambda i: (i, 0)),
        out_shape=jax.ShapeDtypeStruct((B, 1), jnp.float32),
    )(s, gamma.reshape(1, FEAT), beta.reshape(1, FEAT), W1,
      b1.reshape(1, 1024), W2, b2.reshape(1, 512), W3, b3.reshape(1, 1))


def kernel(x, table0, table1, gamma, beta, W1, b1, W2, b2, W3, b3):
    idx = x.astype(jnp.int32).reshape(NW, NCHUNK, H, CHUNK)
    tsum = _add_tables(table0, table1)
    s = _emb_sum(tsum, idx).reshape(B, FEAT)
    return _mlp(s, gamma, beta, W1, b1, W2, b2, W3, b3)
